# Initial kernel scaffold; baseline (speedup 1.0000x reference)
import jax
import jax.numpy as jnp
from jax.experimental import pallas as pl


def kernel(x, source_values, dest_values):
    raise NotImplementedError("write your pallas kernel here")


# speedup vs baseline: 258.1630x; 258.1630x over previous
import functools

import jax
import jax.numpy as jnp
from jax import lax
from jax.experimental import pallas as pl
from jax.experimental.pallas import tpu as pltpu
from jax.experimental.pallas import tpu_sc as plsc

NUM_CORES = 2
NUM_SUBCORES = 16
NUM_WORKERS = NUM_CORES * NUM_SUBCORES
LANES = 16
N_TOTAL = 2 * 1 * 160 * 192 * 224
PER_WORKER = N_TOTAL // NUM_WORKERS
CHUNK = 43008
NUM_CHUNKS = PER_WORKER // CHUNK
TBL = 32


@functools.partial(
    pl.kernel,
    out_type=jax.ShapeDtypeStruct((N_TOTAL,), jnp.float32),
    mesh=plsc.VectorSubcoreMesh(
        core_axis_name="c",
        subcore_axis_name="s",
        num_cores=NUM_CORES,
        num_subcores=NUM_SUBCORES,
    ),
    scratch_types=[
        pltpu.VMEM((TBL,), jnp.float32),
        pltpu.VMEM((CHUNK,), jnp.float32),
        pltpu.VMEM((TBL,), jnp.int32),
        pltpu.VMEM((TBL,), jnp.int32),
    ],
    compiler_params=pltpu.CompilerParams(needs_layout_passes=False),
)
def _remap(x_hbm, src_hbm, dst_hbm, out_hbm, tab_v, buf_v, src_v, dst_v):
    wid = lax.axis_index("s") * NUM_CORES + lax.axis_index("c")
    base = wid * PER_WORKER

    pltpu.sync_copy(src_hbm, src_v)
    pltpu.sync_copy(dst_hbm, dst_v)
    zero = jnp.zeros((LANES,), jnp.float32)
    tab_v[pl.ds(0, LANES)] = zero
    tab_v[pl.ds(LANES, LANES)] = zero
    for h in range(TBL // LANES):
        s = src_v[pl.ds(h * LANES, LANES)]
        d = dst_v[pl.ds(h * LANES, LANES)].astype(jnp.float32)
        plsc.store_scatter(tab_v, [s], d)

    for c in range(NUM_CHUNKS):
        off = base + c * CHUNK
        pltpu.sync_copy(x_hbm.at[pl.ds(off, CHUNK)], buf_v)

        def inner(i, carry):
            idx = buf_v[pl.ds(i * LANES, LANES)].astype(jnp.int32)
            buf_v[pl.ds(i * LANES, LANES)] = plsc.load_gather(tab_v, [idx])
            return carry

        lax.fori_loop(0, CHUNK // LANES, inner, 0)
        pltpu.sync_copy(buf_v, out_hbm.at[pl.ds(off, CHUNK)])


def kernel(x, source_values, dest_values):
    out = _remap(x.reshape(N_TOTAL), source_values, dest_values)
    return out.reshape(x.shape)


# speedup vs baseline: 657.2771x; 2.5460x over previous
import functools

import jax
import jax.numpy as jnp
from jax import lax
from jax.experimental import pallas as pl
from jax.experimental.pallas import tpu as pltpu
from jax.experimental.pallas import tpu_sc as plsc

NUM_CORES = 2
NUM_SUBCORES = 16
NUM_WORKERS = NUM_CORES * NUM_SUBCORES
LANES = 16
N_TOTAL = 2 * 1 * 160 * 192 * 224
PER_WORKER = N_TOTAL // NUM_WORKERS
CHUNK = 21504
NUM_CHUNKS = PER_WORKER // CHUNK
NBUF = 2
TBL = 32
UNROLL = 8


@functools.partial(
    pl.kernel,
    out_type=jax.ShapeDtypeStruct((N_TOTAL,), jnp.float32),
    mesh=plsc.VectorSubcoreMesh(
        core_axis_name="c",
        subcore_axis_name="s",
        num_cores=NUM_CORES,
        num_subcores=NUM_SUBCORES,
    ),
    scratch_types=[
        pltpu.VMEM((TBL,), jnp.float32),
        pltpu.VMEM((TBL,), jnp.int32),
        pltpu.VMEM((TBL,), jnp.int32),
        [pltpu.VMEM((CHUNK,), jnp.float32) for _ in range(NBUF)],
        [pltpu.VMEM((CHUNK,), jnp.float32) for _ in range(NBUF)],
        [pltpu.SemaphoreType.DMA for _ in range(NBUF)],
        [pltpu.SemaphoreType.DMA for _ in range(NBUF)],
    ],
    compiler_params=pltpu.CompilerParams(needs_layout_passes=False),
)
def _remap(x_hbm, src_hbm, dst_hbm, out_hbm, tab_v, src_v, dst_v, inb, outb,
           in_sem, out_sem):
    wid = lax.axis_index("s") * NUM_CORES + lax.axis_index("c")
    base = wid * PER_WORKER

    pltpu.sync_copy(src_hbm, src_v)
    pltpu.sync_copy(dst_hbm, dst_v)
    zero = jnp.zeros((LANES,), jnp.float32)
    tab_v[pl.ds(0, LANES)] = zero
    tab_v[pl.ds(LANES, LANES)] = zero
    for h in range(TBL // LANES):
        s = src_v[pl.ds(h * LANES, LANES)]
        d = dst_v[pl.ds(h * LANES, LANES)].astype(jnp.float32)
        plsc.store_scatter(tab_v, [s], d)

    in_h = [None] * NBUF
    out_h = [None] * NBUF
    for b in range(NBUF):
        off = base + b * CHUNK
        in_h[b] = pltpu.async_copy(x_hbm.at[pl.ds(off, CHUNK)], inb[b],
                                   in_sem[b])

    for c in range(NUM_CHUNKS):
        b = c % NBUF
        off = base + c * CHUNK
        in_h[b].wait()
        if c >= NBUF:
            out_h[b].wait()

        src_buf = inb[b]
        dst_buf = outb[b]

        @plsc.parallel_loop(0, CHUNK, LANES, unroll=UNROLL)
        def _(o):
            idx = src_buf[pl.ds(o, LANES)].astype(jnp.int32)
            dst_buf[pl.ds(o, LANES)] = plsc.load_gather(tab_v, [idx])

        out_h[b] = pltpu.async_copy(dst_buf, out_hbm.at[pl.ds(off, CHUNK)],
                                    out_sem[b])
        nxt = c + NBUF
        if nxt < NUM_CHUNKS:
            in_h[b] = pltpu.async_copy(
                x_hbm.at[pl.ds(base + nxt * CHUNK, CHUNK)], inb[b], in_sem[b])

    for b in range(NBUF):
        out_h[b].wait()


def kernel(x, source_values, dest_values):
    out = _remap(x.reshape(N_TOTAL), source_values, dest_values)
    return out.reshape(x.shape)


# speedup vs baseline: 1456.8311x; 2.2165x over previous
import functools

import jax
import jax.numpy as jnp
from jax import lax
from jax.experimental import pallas as pl
from jax.experimental.pallas import tpu as pltpu
from jax.experimental.pallas import tpu_sc as plsc

NUM_CORES = 2
NUM_SUBCORES = 16
NUM_WORKERS = NUM_CORES * NUM_SUBCORES
LANES = 16
B, D, H, W = 2, 160, 192, 224
NUM_PLANES = B * D
PLANES_PER_WORKER = NUM_PLANES // NUM_WORKERS
VECS_PER_ROW = W // LANES
NBUF = 2
TBL = 32


@functools.partial(
    pl.kernel,
    out_type=jax.ShapeDtypeStruct((B, 1, D, H, W), jnp.float32),
    mesh=plsc.VectorSubcoreMesh(
        core_axis_name="c",
        subcore_axis_name="s",
        num_cores=NUM_CORES,
        num_subcores=NUM_SUBCORES,
    ),
    scratch_types=[
        pltpu.VMEM((TBL,), jnp.float32),
        pltpu.VMEM((TBL,), jnp.int32),
        pltpu.VMEM((TBL,), jnp.int32),
        [pltpu.VMEM((H, W), jnp.float32) for _ in range(NBUF)],
        [pltpu.SemaphoreType.DMA for _ in range(NBUF)],
        [pltpu.SemaphoreType.DMA for _ in range(NBUF)],
    ],
    compiler_params=pltpu.CompilerParams(needs_layout_passes=False),
)
def _remap(x_hbm, src_hbm, dst_hbm, out_hbm, tab_v, src_v, dst_v, buf,
           in_sem, out_sem):
    wid = lax.axis_index("s") * NUM_CORES + lax.axis_index("c")
    plane0 = wid * PLANES_PER_WORKER

    pltpu.sync_copy(src_hbm, src_v)
    pltpu.sync_copy(dst_hbm, dst_v)
    zero = jnp.zeros((LANES,), jnp.float32)
    tab_v[pl.ds(0, LANES)] = zero
    tab_v[pl.ds(LANES, LANES)] = zero
    for h in range(TBL // LANES):
        s = src_v[pl.ds(h * LANES, LANES)]
        d = dst_v[pl.ds(h * LANES, LANES)].astype(jnp.float32)
        plsc.store_scatter(tab_v, [s], d)

    def plane_slice(ref, p):
        n = p // D
        d = p % D
        return ref.at[n, 0, d]

    in_h = [None] * NBUF
    out_h = [None] * NBUF
    for b in range(NBUF):
        in_h[b] = pltpu.async_copy(plane_slice(x_hbm, plane0 + b), buf[b],
                                   in_sem[b])

    for c in range(PLANES_PER_WORKER):
        b = c % NBUF
        in_h[b].wait()

        work = buf[b]

        @plsc.parallel_loop(0, H, 1, unroll=2)
        def _(r):
            for j in range(VECS_PER_ROW):
                v = work[r, pl.ds(j * LANES, LANES)]
                idx = v.astype(jnp.int32)
                work[r, pl.ds(j * LANES, LANES)] = plsc.load_gather(
                    tab_v, [idx])

        out_h[b] = pltpu.async_copy(work, plane_slice(out_hbm, plane0 + c),
                                    out_sem[b])
        nxt = c + NBUF
        if nxt < PLANES_PER_WORKER:
            out_h[b].wait()
            in_h[b] = pltpu.async_copy(plane_slice(x_hbm, plane0 + nxt),
                                       buf[b], in_sem[b])

    out_h[(PLANES_PER_WORKER - 2) % NBUF].wait()
    out_h[(PLANES_PER_WORKER - 1) % NBUF].wait()


def kernel(x, source_values, dest_values):
    return _remap(x, source_values, dest_values)


# speedup vs baseline: 1678.6712x; 1.1523x over previous
import functools

import jax
import jax.numpy as jnp
from jax import lax
from jax.experimental import pallas as pl
from jax.experimental.pallas import tpu as pltpu
from jax.experimental.pallas import tpu_sc as plsc

NUM_CORES = 2
NUM_SUBCORES = 16
NUM_WORKERS = NUM_CORES * NUM_SUBCORES
LANES = 16
B, D, H, W = 2, 160, 192, 224
HH = H // 2
NUM_TILES = B * D * 2
TILES_PER_WORKER = NUM_TILES // NUM_WORKERS
VECS_PER_ROW = W // LANES
NBUF = 2
TBL = 32


@functools.partial(
    pl.kernel,
    out_type=jax.ShapeDtypeStruct((B, 1, D, H, W), jnp.float32),
    mesh=plsc.VectorSubcoreMesh(
        core_axis_name="c",
        subcore_axis_name="s",
        num_cores=NUM_CORES,
        num_subcores=NUM_SUBCORES,
    ),
    scratch_types=[
        pltpu.VMEM((TBL,), jnp.float32),
        pltpu.VMEM((TBL,), jnp.int32),
        pltpu.VMEM((TBL,), jnp.int32),
        [pltpu.VMEM((HH, W), jnp.float32) for _ in range(NBUF)],
        [pltpu.VMEM((HH, W), jnp.float32) for _ in range(NBUF)],
        [pltpu.SemaphoreType.DMA for _ in range(NBUF)],
        [pltpu.SemaphoreType.DMA for _ in range(NBUF)],
    ],
    compiler_params=pltpu.CompilerParams(needs_layout_passes=False),
)
def _remap(x_hbm, src_hbm, dst_hbm, out_hbm, tab_v, src_v, dst_v, inb, outb,
           in_sem, out_sem):
    wid = lax.axis_index("s") * NUM_CORES + lax.axis_index("c")
    tile0 = wid * TILES_PER_WORKER

    pltpu.sync_copy(src_hbm, src_v)
    pltpu.sync_copy(dst_hbm, dst_v)
    zero = jnp.zeros((LANES,), jnp.float32)
    tab_v[pl.ds(0, LANES)] = zero
    tab_v[pl.ds(LANES, LANES)] = zero
    for h in range(TBL // LANES):
        s = src_v[pl.ds(h * LANES, LANES)]
        d = dst_v[pl.ds(h * LANES, LANES)].astype(jnp.float32)
        plsc.store_scatter(tab_v, [s], d)

    def tile_slice(ref, t):
        n = t // (D * 2)
        r = t % (D * 2)
        d = r // 2
        h = (r % 2) * HH
        return ref.at[n, 0, d, pl.ds(h, HH)]

    in_h = [None] * NBUF
    out_h = [None] * NBUF
    for b in range(NBUF):
        in_h[b] = pltpu.async_copy(tile_slice(x_hbm, tile0 + b), inb[b],
                                   in_sem[b])

    for c in range(TILES_PER_WORKER):
        b = c % NBUF
        in_h[b].wait()
        if c >= NBUF:
            out_h[b].wait()

        src_buf = inb[b]
        dst_buf = outb[b]

        @plsc.parallel_loop(0, HH, 1, unroll=2)
        def _(r):
            for j in range(VECS_PER_ROW):
                v = src_buf[r, pl.ds(j * LANES, LANES)]
                idx = v.astype(jnp.int32)
                dst_buf[r, pl.ds(j * LANES, LANES)] = plsc.load_gather(
                    tab_v, [idx])

        out_h[b] = pltpu.async_copy(dst_buf, tile_slice(out_hbm, tile0 + c),
                                    out_sem[b])
        nxt = c + NBUF
        if nxt < TILES_PER_WORKER:
            in_h[b] = pltpu.async_copy(tile_slice(x_hbm, tile0 + nxt),
                                       inb[b], in_sem[b])

    out_h[(TILES_PER_WORKER - 2) % NBUF].wait()
    out_h[(TILES_PER_WORKER - 1) % NBUF].wait()


def kernel(x, source_values, dest_values):
    return _remap(x, source_values, dest_values)


# speedup vs baseline: 1689.4011x; 1.0064x over previous
import functools

import jax
import jax.numpy as jnp
from jax import lax
from jax.experimental import pallas as pl
from jax.experimental.pallas import tpu as pltpu
from jax.experimental.pallas import tpu_sc as plsc

NUM_CORES = 2
NUM_SUBCORES = 16
NUM_WORKERS = NUM_CORES * NUM_SUBCORES
LANES = 16
B, D, H, W = 2, 160, 192, 224
HH = H // 2
NUM_TILES = B * D * 2
TILES_PER_WORKER = NUM_TILES // NUM_WORKERS
VECS_PER_ROW = W // LANES
NBUF = 2
TBL = 32


@functools.partial(
    pl.kernel,
    out_type=jax.ShapeDtypeStruct((B, 1, D, H, W), jnp.float32),
    mesh=plsc.VectorSubcoreMesh(
        core_axis_name="c",
        subcore_axis_name="s",
        num_cores=NUM_CORES,
        num_subcores=NUM_SUBCORES,
    ),
    scratch_types=[
        pltpu.VMEM((TBL, LANES), jnp.float32),
        pltpu.VMEM((TBL,), jnp.int32),
        pltpu.VMEM((TBL,), jnp.int32),
        [pltpu.VMEM((HH, W), jnp.float32) for _ in range(NBUF)],
        [pltpu.VMEM((HH, W), jnp.float32) for _ in range(NBUF)],
        [pltpu.SemaphoreType.DMA for _ in range(NBUF)],
        [pltpu.SemaphoreType.DMA for _ in range(NBUF)],
    ],
    compiler_params=pltpu.CompilerParams(needs_layout_passes=False),
)
def _remap(x_hbm, src_hbm, dst_hbm, out_hbm, tab_v, src_v, dst_v, inb, outb,
           in_sem, out_sem):
    wid = lax.axis_index("s") * NUM_CORES + lax.axis_index("c")
    tile0 = wid * TILES_PER_WORKER

    pltpu.sync_copy(src_hbm, src_v)
    pltpu.sync_copy(dst_hbm, dst_v)
    lane = lax.iota(jnp.int32, LANES)
    zero = jnp.zeros((LANES,), jnp.float32)
    for v in range(TBL):
        tab_v[v, pl.ds(0, LANES)] = zero
    for h in range(TBL // LANES):
        s = src_v[pl.ds(h * LANES, LANES)]
        d = dst_v[pl.ds(h * LANES, LANES)].astype(jnp.float32)
        for l in range(LANES):
            plsc.store_scatter(tab_v, [s, jnp.full((LANES,), l, jnp.int32)],
                               d)

    def tile_slice(ref, t):
        n = t // (D * 2)
        r = t % (D * 2)
        d = r // 2
        h = (r % 2) * HH
        return ref.at[n, 0, d, pl.ds(h, HH)]

    in_h = [None] * NBUF
    out_h = [None] * NBUF
    for b in range(NBUF):
        in_h[b] = pltpu.async_copy(tile_slice(x_hbm, tile0 + b), inb[b],
                                   in_sem[b])

    for c in range(TILES_PER_WORKER):
        b = c % NBUF
        in_h[b].wait()
        if c >= NBUF:
            out_h[b].wait()

        src_buf = inb[b]
        dst_buf = outb[b]

        @plsc.parallel_loop(0, HH, 1, unroll=2)
        def _(r):
            for j in range(VECS_PER_ROW):
                v = src_buf[r, pl.ds(j * LANES, LANES)]
                idx = v.astype(jnp.int32)
                dst_buf[r, pl.ds(j * LANES, LANES)] = plsc.load_gather(
                    tab_v, [idx, lane])

        out_h[b] = pltpu.async_copy(dst_buf, tile_slice(out_hbm, tile0 + c),
                                    out_sem[b])
        nxt = c + NBUF
        if nxt < TILES_PER_WORKER:
            in_h[b] = pltpu.async_copy(tile_slice(x_hbm, tile0 + nxt),
                                       inb[b], in_sem[b])

    out_h[(TILES_PER_WORKER - 2) % NBUF].wait()
    out_h[(TILES_PER_WORKER - 1) % NBUF].wait()


def kernel(x, source_values, dest_values):
    return _remap(x, source_values, dest_values)


# speedup vs baseline: 1911.9297x; 1.1317x over previous
import functools

import jax
import jax.numpy as jnp
from jax import lax
from jax.experimental import pallas as pl
from jax.experimental.pallas import tpu as pltpu
from jax.experimental.pallas import tpu_sc as plsc

NUM_CORES = 2
NUM_SUBCORES = 16
NUM_WORKERS = NUM_CORES * NUM_SUBCORES
LANES = 16
B, D, H, W = 2, 160, 192, 224
SPLIT = 4
HH = H // SPLIT
NUM_TILES = B * D * SPLIT
TILES_PER_WORKER = NUM_TILES // NUM_WORKERS
VECS_PER_ROW = W // LANES
NBUF = 4
TBL = 32


@functools.partial(
    pl.kernel,
    out_type=jax.ShapeDtypeStruct((B, 1, D, H, W), jnp.float32),
    mesh=plsc.VectorSubcoreMesh(
        core_axis_name="c",
        subcore_axis_name="s",
        num_cores=NUM_CORES,
        num_subcores=NUM_SUBCORES,
    ),
    scratch_types=[
        pltpu.VMEM((TBL, LANES), jnp.float32),
        pltpu.VMEM((TBL,), jnp.int32),
        pltpu.VMEM((TBL,), jnp.int32),
        [pltpu.VMEM((HH, W), jnp.float32) for _ in range(NBUF)],
        [pltpu.VMEM((HH, W), jnp.float32) for _ in range(NBUF)],
        [pltpu.SemaphoreType.DMA for _ in range(NBUF)],
        [pltpu.SemaphoreType.DMA for _ in range(NBUF)],
    ],
    compiler_params=pltpu.CompilerParams(needs_layout_passes=False),
)
def _remap(x_hbm, src_hbm, dst_hbm, out_hbm, tab_v, src_v, dst_v, inb, outb,
           in_sem, out_sem):
    wid = lax.axis_index("s") * NUM_CORES + lax.axis_index("c")
    tile0 = wid * TILES_PER_WORKER

    pltpu.sync_copy(src_hbm, src_v)
    pltpu.sync_copy(dst_hbm, dst_v)
    lane = lax.iota(jnp.int32, LANES)
    zero = jnp.zeros((LANES,), jnp.float32)
    for v in range(TBL):
        tab_v[v, pl.ds(0, LANES)] = zero
    for h in range(TBL // LANES):
        s = src_v[pl.ds(h * LANES, LANES)]
        d = dst_v[pl.ds(h * LANES, LANES)].astype(jnp.float32)
        for l in range(LANES):
            plsc.store_scatter(tab_v, [s, jnp.full((LANES,), l, jnp.int32)],
                               d)

    def tile_slice(ref, t):
        n = t // (D * SPLIT)
        r = t % (D * SPLIT)
        d = r // SPLIT
        h = (r % SPLIT) * HH
        return ref.at[n, 0, d, pl.ds(h, HH)]

    for b in range(NBUF):
        pltpu.async_copy(tile_slice(x_hbm, tile0 + b), inb[b], in_sem[b])

    @pl.loop(0, TILES_PER_WORKER, step=NBUF)
    def _(c0):
        for b in range(NBUF):
            c = c0 + b
            t = tile0 + c
            pltpu.make_async_copy(tile_slice(x_hbm, t), inb[b],
                                  in_sem[b]).wait()

            @pl.when(c >= NBUF)
            def _():
                pltpu.make_async_copy(outb[b], tile_slice(out_hbm, t),
                                      out_sem[b]).wait()

            src_buf = inb[b]
            dst_buf = outb[b]

            @plsc.parallel_loop(0, HH, 1, unroll=2)
            def _(r):
                for j in range(VECS_PER_ROW):
                    v = src_buf[r, pl.ds(j * LANES, LANES)]
                    idx = v.astype(jnp.int32)
                    dst_buf[r, pl.ds(j * LANES, LANES)] = plsc.load_gather(
                        tab_v, [idx, lane])

            pltpu.async_copy(dst_buf, tile_slice(out_hbm, t), out_sem[b])

            @pl.when(c + NBUF < TILES_PER_WORKER)
            def _():
                pltpu.async_copy(tile_slice(x_hbm, t + NBUF), inb[b],
                                 in_sem[b])

    for b in range(NBUF):
        t = tile0 + TILES_PER_WORKER - NBUF + b
        pltpu.make_async_copy(outb[b], tile_slice(out_hbm, t),
                              out_sem[b]).wait()


def kernel(x, source_values, dest_values):
    return _remap(x, source_values, dest_values)
